# Initial kernel scaffold; baseline (speedup 1.0000x reference)
import jax
import jax.numpy as jnp
from jax.experimental import pallas as pl


def kernel(x, params, edge_indices, edge_masks):
    raise NotImplementedError("write your pallas kernel here")


# speedup vs baseline: 1.0670x; 1.0670x over previous
import functools

import jax
import jax.numpy as jnp
import numpy as np
from jax.experimental import pallas as pl
from jax.experimental.pallas import tpu as pltpu

_N = 90
_NP = 96
_C = 96
_CP = 128
_E = 5
_NB = 6
_HD = 24
_BT = 16
_ASP = 2086
_ASPP = 2176
_PREC = jax.lax.Precision.HIGHEST


def _pad2(a, r, c):
    return jnp.pad(a, ((0, r - a.shape[0]), (0, c - a.shape[1])))


def _trunk_body(x_r, adj_r, inw_r, inb_r, m_r, wp_r, bp_r, lns_r, lnb_r,
                w1_r, b1_r, w2_r, b2_r, s1w_r, s1b_r, s2w_r, s2b_r,
                p1w_r, p1b_r, v1w_r, v1b_r, v2w_r, v2b_r,
                m1w_r, m1b_r, m2w_r, m2b_r,
                p32_r, vq_r, mat_r):
    f32 = jnp.float32
    R = _BT * _NP
    dot = functools.partial(jax.lax.dot_general, precision=_PREC,
                            preferred_element_type=f32)

    def mm(a, b):
        return dot(a, b, (((1,), (0,)), ((), ())))

    def bmm(a, b, ca, cb):
        return dot(a, b, (((ca,), (cb,)), ((0,), (0,))))

    ones_col = (jax.lax.broadcasted_iota(jnp.int32, (1, _CP), 1) == _C).astype(f32)
    fmask = (jax.lax.broadcasted_iota(jnp.int32, (1, _CP), 1) < _C).astype(f32)
    rmask3 = (jax.lax.broadcasted_iota(jnp.int32, (1, _NP, 1), 1) < _N).astype(f32)

    h = jnp.maximum(mm(x_r[...], inw_r[...]) + inb_r[...], 0.0)
    adj_all = adj_r[...]

    for b in range(_NB):
        hA = h + ones_col
        t_all = mm(hA, m_r[b])
        hp_all = mm(h, wp_r[b])
        h3 = h.reshape(_BT, _NP, _CP)
        hA3 = hA.reshape(_BT, _NP, _CP)
        agg3 = jnp.zeros((_BT, _NP, _CP), f32)
        for i in range(_E):
            t3 = t_all[:, i * _CP:(i + 1) * _CP].reshape(_BT, _NP, _CP)
            lg = bmm(t3, hA3, 2, 2)
            am = adj_all[i][None] > 0.0
            lg = jnp.where(am, lg, -1e9)
            w = jax.nn.softmax(lg, axis=-1)
            w = jnp.where(am, w, 0.0)
            hp3 = hp_all[:, i * _CP:(i + 1) * _CP].reshape(_BT, _NP, _CP)
            agg3 = agg3 + bmm(w, hp3, 2, 1)
        agg = agg3.reshape(R, _CP) + bp_r[b]
        gmean = jnp.sum(h3 * rmask3, axis=1) * (1.0 / _N)
        gb = jnp.broadcast_to(gmean[:, None, :], (_BT, _NP, _CP)).reshape(R, _CP)
        xcat = jnp.concatenate([h, agg, gb], axis=1)
        mu = jnp.sum(xcat, axis=1, keepdims=True) * (1.0 / (3 * _C))
        ex2 = jnp.sum(xcat * xcat, axis=1, keepdims=True) * (1.0 / (3 * _C))
        inv = jax.lax.rsqrt(ex2 - mu * mu + 1e-6)
        xh = (xcat - mu) * inv * lns_r[b] + lnb_r[b]
        y = jnp.maximum(mm(xh, w1_r[b]) + b1_r[b], 0.0)
        y = mm(y, w2_r[b]) + b2_r[b]
        out = h + y
        o3 = out.reshape(_BT, _NP, _CP)
        om = jnp.sum(o3 * rmask3, axis=1) * (1.0 / _N)
        s1 = jnp.maximum(mm(om, s1w_r[b]) + s1b_r[b], 0.0)
        s2 = jax.nn.sigmoid(mm(s1, s2w_r[b]) + s2b_r[b]) * fmask
        h = (o3 * s2[:, None, :]).reshape(R, _CP)

    p32_r[...] = jnp.maximum(mm(h, p1w_r[...]) + p1b_r[...], 0.0)
    h3 = h.reshape(_BT, _NP, _CP)
    hm = jnp.sum(h3 * rmask3, axis=1) * (1.0 / _N)
    v = jnp.maximum(mm(hm, v1w_r[...]) + v1b_r[...], 0.0)
    vq_r[...] = jnp.tanh(mm(v, v2w_r[...]) + v2b_r[...])
    mr = jnp.maximum(mm(hm, m1w_r[...]) + m1b_r[...], 0.0)
    mat_r[...] = jnp.tanh(jnp.sum(mr * m2w_r[...], axis=1, keepdims=True)
                          + m2b_r[...])


def _policy_body(x_r, w_r, b_r, o_r):
    o_r[...] = jax.lax.dot_general(
        x_r[...], w_r[...], (((1,), (0,)), ((), ())),
        precision=_PREC, preferred_element_type=jnp.float32) + b_r[...]


def kernel(x, params, edge_indices, edge_masks):
    p = params
    B = x.shape[0]
    gB = B // _BT
    f32 = jnp.float32

    x2 = jnp.transpose(x, (0, 2, 3, 1)).reshape(B, _N, -1)
    inch = x2.shape[-1]
    x2 = jnp.pad(x2, ((0, 0), (0, _NP - _N), (0, 16 - inch)))
    x2 = x2.reshape(B * _NP, 16)

    inw = _pad2(p['in_proj']['w'], 16, _CP)
    inb = _pad2(p['in_proj']['b'][None, :], 1, _CP)

    adjs = []
    for i in range(_E):
        idx = jnp.where(edge_indices[i] < 0, 0, edge_indices[i])
        a = jnp.sum(jax.nn.one_hot(idx, _N, dtype=f32)
                    * edge_masks[i][..., None], axis=1)
        adjs.append(_pad2(a, _NP, _NP))
    adj = jnp.stack(adjs)

    def seg3(v, fill):
        segs = [jnp.pad(v[j * _C:(j + 1) * _C], (0, _CP - _C),
                        constant_values=fill) for j in range(3)]
        return jnp.concatenate(segs)[None, :]

    Ms, Wps, bps, lnss, lnbs = [], [], [], [], []
    w1s, b1s, w2s, b2s = [], [], [], []
    s1ws, s1bs, s2ws, s2bs = [], [], [], []
    for blk in p['blocks']:
        mi, wpi, bsum = [], [], 0.0
        for i in range(_E):
            qa = _pad2(jnp.concatenate(
                [blk['attn_q'][i]['w'], blk['attn_q'][i]['b'][None, :]], 0),
                _CP, _HD)
            ka = _pad2(jnp.concatenate(
                [blk['attn_k'][i]['w'], blk['attn_k'][i]['b'][None, :]], 0),
                _CP, _HD)
            mi.append(jnp.dot(qa, ka.T, precision=_PREC) * (1.0 / np.sqrt(_HD)))
            wpi.append(_pad2(blk['edge_proj'][i]['w'], _CP, _CP))
            bsum = bsum + blk['edge_proj'][i]['b']
        Ms.append(jnp.concatenate(mi, axis=1))
        Wps.append(jnp.concatenate(wpi, axis=1))
        bps.append(_pad2(bsum[None, :], 1, _CP))
        lnss.append(seg3(blk['ln']['scale'], 0.0))
        lnbs.append(seg3(blk['ln']['bias'], 0.0))
        w1 = blk['mlp1']['w']
        w1s.append(jnp.concatenate(
            [_pad2(w1[j * _C:(j + 1) * _C], _CP, _CP) for j in range(3)], 0))
        b1s.append(_pad2(blk['mlp1']['b'][None, :], 1, _CP))
        w2s.append(_pad2(blk['mlp2']['w'], _CP, _CP))
        b2s.append(_pad2(blk['mlp2']['b'][None, :], 1, _CP))
        s1ws.append(_pad2(blk['se1']['w'], _CP, _HD))
        s1bs.append(blk['se1']['b'][None, :])
        s2ws.append(_pad2(blk['se2']['w'], _HD, _CP))
        s2bs.append(_pad2(blk['se2']['b'][None, :], 1, _CP))

    m6 = jnp.stack(Ms)
    wp6 = jnp.stack(Wps)
    bp6 = jnp.stack(bps)
    lns6 = jnp.stack(lnss)
    lnb6 = jnp.stack(lnbs)
    w16 = jnp.stack(w1s)
    b16 = jnp.stack(b1s)
    w26 = jnp.stack(w2s)
    b26 = jnp.stack(b2s)
    s1w6 = jnp.stack(s1ws)
    s1b6 = jnp.stack(s1bs)
    s2w6 = jnp.stack(s2ws)
    s2b6 = jnp.stack(s2bs)

    p1w = _pad2(p['policy1']['w'], _CP, 32)
    p1b = p['policy1']['b'][None, :]
    v1w = _pad2(p['value1']['w'], _CP, _CP)
    v1b = p['value1']['b'][None, :]
    v2w = p['value2']['w']
    v2b = p['value2']['b'][None, :]
    m1w = _pad2(p['mat1']['w'], _CP, 64)
    m1b = p['mat1']['b'][None, :]
    m2w = p['mat2']['w'].T
    m2b = p['mat2']['b'][None, :]

    def const_spec(shape):
        n = len(shape)
        return pl.BlockSpec(shape, lambda i, _n=n: (0,) * _n)

    R = _BT * _NP
    in_specs = [
        pl.BlockSpec((R, 16), lambda i: (i, 0)),
        const_spec(adj.shape),
        const_spec(inw.shape), const_spec(inb.shape),
        const_spec(m6.shape), const_spec(wp6.shape), const_spec(bp6.shape),
        const_spec(lns6.shape), const_spec(lnb6.shape),
        const_spec(w16.shape), const_spec(b16.shape),
        const_spec(w26.shape), const_spec(b26.shape),
        const_spec(s1w6.shape), const_spec(s1b6.shape),
        const_spec(s2w6.shape), const_spec(s2b6.shape),
        const_spec(p1w.shape), const_spec(p1b.shape),
        const_spec(v1w.shape), const_spec(v1b.shape),
        const_spec(v2w.shape), const_spec(v2b.shape),
        const_spec(m1w.shape), const_spec(m1b.shape),
        const_spec(m2w.shape), const_spec(m2b.shape),
    ]
    out_specs = [
        pl.BlockSpec((R, 32), lambda i: (i, 0)),
        pl.BlockSpec((_BT, 64), lambda i: (i, 0)),
        pl.BlockSpec((_BT, 1), lambda i: (i, 0)),
    ]
    out_shapes = [
        jax.ShapeDtypeStruct((B * _NP, 32), f32),
        jax.ShapeDtypeStruct((B, 64), f32),
        jax.ShapeDtypeStruct((B, 1), f32),
    ]
    p32, vq, mat = pl.pallas_call(
        _trunk_body,
        grid=(gB,),
        in_specs=in_specs,
        out_specs=out_specs,
        out_shape=out_shapes,
        compiler_params=pltpu.CompilerParams(
            dimension_semantics=("arbitrary",)),
    )(x2, adj, inw, inb, m6, wp6, bp6, lns6, lnb6, w16, b16, w26, b26,
      s1w6, s1b6, s2w6, s2b6, p1w, p1b, v1w, v1b, v2w, v2b,
      m1w, m1b, m2w, m2b)

    pflat = p32.reshape(B, _NP, 32)[:, :_N, :].reshape(B, _N * 32)
    pw = _pad2(p['policy2']['w'], _N * 32, _ASPP)
    pb = _pad2(p['policy2']['b'][None, :], 1, _ASPP)
    nct = _ASPP // _CP
    pol = pl.pallas_call(
        _policy_body,
        grid=(nct,),
        in_specs=[
            pl.BlockSpec((B, _N * 32), lambda j: (0, 0)),
            pl.BlockSpec((_N * 32, _CP), lambda j: (0, j)),
            pl.BlockSpec((1, _CP), lambda j: (0, j)),
        ],
        out_specs=pl.BlockSpec((B, _CP), lambda j: (0, j)),
        out_shape=jax.ShapeDtypeStruct((B, _ASPP), f32),
        compiler_params=pltpu.CompilerParams(
            dimension_semantics=("arbitrary",)),
    )(pflat, pw, pb)

    return (pol[:, :_ASP], vq, mat[:, 0])


# speedup vs baseline: 1.1270x; 1.0562x over previous
import functools

import jax
import jax.numpy as jnp
import numpy as np
from jax.experimental import pallas as pl
from jax.experimental.pallas import tpu as pltpu

_N = 90
_NP = 96
_C = 96
_CP = 128
_E = 5
_NB = 6
_HD = 24
_BT = 32
_ASP = 2086
_ASPP = 2176
_PREC = jax.lax.Precision.HIGHEST


def _pad2(a, r, c):
    return jnp.pad(a, ((0, r - a.shape[0]), (0, c - a.shape[1])))


def _trunk_body(x_r, adj_r, inw_r, inb_r, m_r, wp_r, bp_r, lns_r, lnb_r,
                w1_r, b1_r, w2_r, b2_r, s1w_r, s1b_r, s2w_r, s2b_r,
                p1w_r, p1b_r, v1w_r, v1b_r, v2w_r, v2b_r,
                m1w_r, m1b_r, m2w_r, m2b_r,
                p32_r, vq_r, mat_r):
    f32 = jnp.float32
    R = _BT * _NP
    dot = functools.partial(jax.lax.dot_general, precision=_PREC,
                            preferred_element_type=f32)

    def mm(a, b):
        return dot(a, b, (((1,), (0,)), ((), ())))

    def bmm(a, b, ca, cb):
        return dot(a, b, (((ca,), (cb,)), ((0,), (0,))))

    ones_col = (jax.lax.broadcasted_iota(jnp.int32, (1, _CP), 1) == _C).astype(f32)
    fmask = (jax.lax.broadcasted_iota(jnp.int32, (1, _CP), 1) < _C).astype(f32)
    rmask3 = (jax.lax.broadcasted_iota(jnp.int32, (1, _NP, 1), 1) < _N).astype(f32)

    h = jnp.maximum(mm(x_r[...], inw_r[...]) + inb_r[...], 0.0)
    adj_all = adj_r[...]

    for b in range(_NB):
        hA = h + ones_col
        t_all = mm(hA, m_r[b])
        hp_all = mm(h, wp_r[b])
        h3 = h.reshape(_BT, _NP, _CP)
        hA3 = hA.reshape(_BT, _NP, _CP)
        agg3 = jnp.zeros((_BT, _NP, _CP), f32)
        for i in range(_E):
            t3 = t_all[:, i * _CP:(i + 1) * _CP].reshape(_BT, _NP, _CP)
            lg = bmm(t3, hA3, 2, 2)
            am = adj_all[i][None] > 0.0
            lg = jnp.where(am, lg, -1e9)
            w = jax.nn.softmax(lg, axis=-1)
            w = jnp.where(am, w, 0.0)
            hp3 = hp_all[:, i * _CP:(i + 1) * _CP].reshape(_BT, _NP, _CP)
            agg3 = agg3 + bmm(w, hp3, 2, 1)
        agg = agg3.reshape(R, _CP) + bp_r[b]
        gmean = jnp.sum(h3 * rmask3, axis=1) * (1.0 / _N)
        gb = jnp.broadcast_to(gmean[:, None, :], (_BT, _NP, _CP)).reshape(R, _CP)
        xcat = jnp.concatenate([h, agg, gb], axis=1)
        mu = jnp.sum(xcat, axis=1, keepdims=True) * (1.0 / (3 * _C))
        ex2 = jnp.sum(xcat * xcat, axis=1, keepdims=True) * (1.0 / (3 * _C))
        inv = jax.lax.rsqrt(ex2 - mu * mu + 1e-6)
        xh = (xcat - mu) * inv * lns_r[b] + lnb_r[b]
        y = jnp.maximum(mm(xh, w1_r[b]) + b1_r[b], 0.0)
        y = mm(y, w2_r[b]) + b2_r[b]
        out = h + y
        o3 = out.reshape(_BT, _NP, _CP)
        om = jnp.sum(o3 * rmask3, axis=1) * (1.0 / _N)
        s1 = jnp.maximum(mm(om, s1w_r[b]) + s1b_r[b], 0.0)
        s2 = jax.nn.sigmoid(mm(s1, s2w_r[b]) + s2b_r[b]) * fmask
        h = (o3 * s2[:, None, :]).reshape(R, _CP)

    p32_r[...] = jnp.maximum(mm(h, p1w_r[...]) + p1b_r[...], 0.0)
    h3 = h.reshape(_BT, _NP, _CP)
    hm = jnp.sum(h3 * rmask3, axis=1) * (1.0 / _N)
    v = jnp.maximum(mm(hm, v1w_r[...]) + v1b_r[...], 0.0)
    vq_r[...] = jnp.tanh(mm(v, v2w_r[...]) + v2b_r[...])
    mr = jnp.maximum(mm(hm, m1w_r[...]) + m1b_r[...], 0.0)
    mat_r[...] = jnp.tanh(jnp.sum(mr * m2w_r[...], axis=1, keepdims=True)
                          + m2b_r[...])


def _policy_body(x_r, w_r, b_r, o_r):
    o_r[...] = jax.lax.dot_general(
        x_r[...], w_r[...], (((1,), (0,)), ((), ())),
        precision=_PREC, preferred_element_type=jnp.float32) + b_r[...]


def kernel(x, params, edge_indices, edge_masks):
    p = params
    B = x.shape[0]
    gB = B // _BT
    f32 = jnp.float32

    x2 = jnp.transpose(x, (0, 2, 3, 1)).reshape(B, _N, -1)
    inch = x2.shape[-1]
    x2 = jnp.pad(x2, ((0, 0), (0, _NP - _N), (0, 16 - inch)))
    x2 = x2.reshape(B * _NP, 16)

    inw = _pad2(p['in_proj']['w'], 16, _CP)
    inb = _pad2(p['in_proj']['b'][None, :], 1, _CP)

    adjs = []
    for i in range(_E):
        idx = jnp.where(edge_indices[i] < 0, 0, edge_indices[i])
        a = jnp.sum(jax.nn.one_hot(idx, _N, dtype=f32)
                    * edge_masks[i][..., None], axis=1)
        adjs.append(_pad2(a, _NP, _NP))
    adj = jnp.stack(adjs)

    def seg3(v, fill):
        segs = [jnp.pad(v[j * _C:(j + 1) * _C], (0, _CP - _C),
                        constant_values=fill) for j in range(3)]
        return jnp.concatenate(segs)[None, :]

    Ms, Wps, bps, lnss, lnbs = [], [], [], [], []
    w1s, b1s, w2s, b2s = [], [], [], []
    s1ws, s1bs, s2ws, s2bs = [], [], [], []
    for blk in p['blocks']:
        mi, wpi, bsum = [], [], 0.0
        for i in range(_E):
            qa = _pad2(jnp.concatenate(
                [blk['attn_q'][i]['w'], blk['attn_q'][i]['b'][None, :]], 0),
                _CP, _HD)
            ka = _pad2(jnp.concatenate(
                [blk['attn_k'][i]['w'], blk['attn_k'][i]['b'][None, :]], 0),
                _CP, _HD)
            mi.append(jnp.dot(qa, ka.T, precision=_PREC) * (1.0 / np.sqrt(_HD)))
            wpi.append(_pad2(blk['edge_proj'][i]['w'], _CP, _CP))
            bsum = bsum + blk['edge_proj'][i]['b']
        Ms.append(jnp.concatenate(mi, axis=1))
        Wps.append(jnp.concatenate(wpi, axis=1))
        bps.append(_pad2(bsum[None, :], 1, _CP))
        lnss.append(seg3(blk['ln']['scale'], 0.0))
        lnbs.append(seg3(blk['ln']['bias'], 0.0))
        w1 = blk['mlp1']['w']
        w1s.append(jnp.concatenate(
            [_pad2(w1[j * _C:(j + 1) * _C], _CP, _CP) for j in range(3)], 0))
        b1s.append(_pad2(blk['mlp1']['b'][None, :], 1, _CP))
        w2s.append(_pad2(blk['mlp2']['w'], _CP, _CP))
        b2s.append(_pad2(blk['mlp2']['b'][None, :], 1, _CP))
        s1ws.append(_pad2(blk['se1']['w'], _CP, _HD))
        s1bs.append(blk['se1']['b'][None, :])
        s2ws.append(_pad2(blk['se2']['w'], _HD, _CP))
        s2bs.append(_pad2(blk['se2']['b'][None, :], 1, _CP))

    m6 = jnp.stack(Ms)
    wp6 = jnp.stack(Wps)
    bp6 = jnp.stack(bps)
    lns6 = jnp.stack(lnss)
    lnb6 = jnp.stack(lnbs)
    w16 = jnp.stack(w1s)
    b16 = jnp.stack(b1s)
    w26 = jnp.stack(w2s)
    b26 = jnp.stack(b2s)
    s1w6 = jnp.stack(s1ws)
    s1b6 = jnp.stack(s1bs)
    s2w6 = jnp.stack(s2ws)
    s2b6 = jnp.stack(s2bs)

    p1w = _pad2(p['policy1']['w'], _CP, 32)
    p1b = p['policy1']['b'][None, :]
    v1w = _pad2(p['value1']['w'], _CP, _CP)
    v1b = p['value1']['b'][None, :]
    v2w = p['value2']['w']
    v2b = p['value2']['b'][None, :]
    m1w = _pad2(p['mat1']['w'], _CP, 64)
    m1b = p['mat1']['b'][None, :]
    m2w = p['mat2']['w'].T
    m2b = p['mat2']['b'][None, :]

    def const_spec(shape):
        n = len(shape)
        return pl.BlockSpec(shape, lambda i, _n=n: (0,) * _n)

    R = _BT * _NP
    in_specs = [
        pl.BlockSpec((R, 16), lambda i: (i, 0)),
        const_spec(adj.shape),
        const_spec(inw.shape), const_spec(inb.shape),
        const_spec(m6.shape), const_spec(wp6.shape), const_spec(bp6.shape),
        const_spec(lns6.shape), const_spec(lnb6.shape),
        const_spec(w16.shape), const_spec(b16.shape),
        const_spec(w26.shape), const_spec(b26.shape),
        const_spec(s1w6.shape), const_spec(s1b6.shape),
        const_spec(s2w6.shape), const_spec(s2b6.shape),
        const_spec(p1w.shape), const_spec(p1b.shape),
        const_spec(v1w.shape), const_spec(v1b.shape),
        const_spec(v2w.shape), const_spec(v2b.shape),
        const_spec(m1w.shape), const_spec(m1b.shape),
        const_spec(m2w.shape), const_spec(m2b.shape),
    ]
    out_specs = [
        pl.BlockSpec((R, 32), lambda i: (i, 0)),
        pl.BlockSpec((_BT, 64), lambda i: (i, 0)),
        pl.BlockSpec((_BT, 1), lambda i: (i, 0)),
    ]
    out_shapes = [
        jax.ShapeDtypeStruct((B * _NP, 32), f32),
        jax.ShapeDtypeStruct((B, 64), f32),
        jax.ShapeDtypeStruct((B, 1), f32),
    ]
    p32, vq, mat = pl.pallas_call(
        _trunk_body,
        grid=(gB,),
        in_specs=in_specs,
        out_specs=out_specs,
        out_shape=out_shapes,
        compiler_params=pltpu.CompilerParams(
            dimension_semantics=("arbitrary",)),
    )(x2, adj, inw, inb, m6, wp6, bp6, lns6, lnb6, w16, b16, w26, b26,
      s1w6, s1b6, s2w6, s2b6, p1w, p1b, v1w, v1b, v2w, v2b,
      m1w, m1b, m2w, m2b)

    pflat = p32.reshape(B, _NP, 32)[:, :_N, :].reshape(B, _N * 32)
    pw = _pad2(p['policy2']['w'], _N * 32, _ASPP)
    pb = _pad2(p['policy2']['b'][None, :], 1, _ASPP)
    nct = _ASPP // _CP
    pol = pl.pallas_call(
        _policy_body,
        grid=(nct,),
        in_specs=[
            pl.BlockSpec((B, _N * 32), lambda j: (0, 0)),
            pl.BlockSpec((_N * 32, _CP), lambda j: (0, j)),
            pl.BlockSpec((1, _CP), lambda j: (0, j)),
        ],
        out_specs=pl.BlockSpec((B, _CP), lambda j: (0, j)),
        out_shape=jax.ShapeDtypeStruct((B, _ASPP), f32),
        compiler_params=pltpu.CompilerParams(
            dimension_semantics=("arbitrary",)),
    )(pflat, pw, pb)

    return (pol[:, :_ASP], vq, mat[:, 0])
